# baseline (speedup 1.0000x reference)
import functools

import jax
import jax.numpy as jnp
from jax import lax
from jax.experimental import pallas as pl
from jax.experimental.pallas import tpu as pltpu
from jax.experimental.pallas import tpu_sc as plsc

N = 10000
E = 320000
F = 128
H = 128

NC = 2
NS = 16
NW = NC * NS
GSZ = 128
RPT = 632
NPAD = NS * RPT

K = 4
ECH = E // K


@functools.cache
def _build_sc_kernels(ngrp):
    mesh = plsc.VectorSubcoreMesh(core_axis_name="c", subcore_axis_name="s",
                                  num_cores=NC, num_subcores=NS)
    ec = ngrp * GSZ
    nseq = (ngrp + NW - 1) // NW
    nit = (nseq + 1) // 2

    dma = pltpu.SemaphoreType.DMA

    @functools.partial(
        pl.kernel,
        out_type=jax.ShapeDtypeStruct((ec, H), jnp.float32),
        mesh=mesh,
        scratch_types=[
            pltpu.VMEM((4, GSZ), jnp.int32),
            pltpu.VMEM((4, GSZ, H), jnp.float32),
            [dma, dma, dma, dma],
            [dma, dma, dma, dma],
            [dma, dma],
        ],
    )
    def sc_gather(table_hbm, idx_hbm, out_hbm, idx_v, rows_v,
                  isems, wsems, gsems):
        c = lax.axis_index("c")
        s = lax.axis_index("s")
        w = c * NS + s

        def idx_load(g, slot):
            pltpu.async_copy(idx_hbm.at[pl.ds(g * GSZ, GSZ)],
                             idx_v.at[slot], isems[slot])

        def idx_wait(g, slot):
            pltpu.make_async_copy(idx_hbm.at[pl.ds(g * GSZ, GSZ)],
                                  idx_v.at[slot], isems[slot]).wait()

        def wout(g, slot):
            pltpu.async_copy(rows_v.at[slot],
                             out_hbm.at[pl.ds(g * GSZ, GSZ), :], wsems[slot])

        def wout_wait(g, slot):
            pltpu.make_async_copy(rows_v.at[slot],
                                  out_hbm.at[pl.ds(g * GSZ, GSZ), :],
                                  wsems[slot]).wait()

        @pl.when(w < ngrp)
        def _():
            idx_load(w, 0)

        @pl.when(w + NW < ngrp)
        def _():
            idx_load(w + NW, 1)

        def body(i, carry):
            g0 = w + 2 * i * NW
            g1 = g0 + NW

            for p in (0, 1):
                @pl.when(lax.rem(i, 2) == p)
                def _():
                    s0, s1 = 2 * p, 2 * p + 1
                    q0, q1 = 2 * (1 - p), 2 * (1 - p) + 1
                    n0 = g0 + 2 * NW
                    n1 = g0 + 3 * NW

                    @pl.when(n0 < ngrp)
                    def _():
                        idx_load(n0, q0)

                    @pl.when(n1 < ngrp)
                    def _():
                        idx_load(n1, q1)

                    @pl.when(g0 < ngrp)
                    def _():
                        idx_wait(g0, s0)

                        @pl.when(i >= 2)
                        def _():
                            wout_wait(g0, s0)
                        pltpu.async_copy(table_hbm.at[idx_v.at[s0]],
                                         rows_v.at[s0], gsems[0])

                    @pl.when(g1 < ngrp)
                    def _():
                        idx_wait(g1, s1)

                        @pl.when(i >= 2)
                        def _():
                            wout_wait(g1, s1)
                        pltpu.async_copy(table_hbm.at[idx_v.at[s1]],
                                         rows_v.at[s1], gsems[1])

                    @pl.when(g0 < ngrp)
                    def _():
                        pltpu.make_async_copy(table_hbm.at[idx_v.at[s0]],
                                              rows_v.at[s0], gsems[0]).wait()
                        wout(g0, s0)

                    @pl.when(g1 < ngrp)
                    def _():
                        pltpu.make_async_copy(table_hbm.at[idx_v.at[s1]],
                                              rows_v.at[s1], gsems[1]).wait()
                        wout(g1, s1)

            return carry

        lax.fori_loop(0, nit, body, 0)
        for slot in range(4):
            wout_wait(0, slot)

    @functools.partial(
        pl.kernel,
        out_type=jax.ShapeDtypeStruct((NC, NPAD, H), jnp.float32),
        mesh=mesh,
        scratch_types=[
            pltpu.VMEM((3, GSZ), jnp.int32),
            pltpu.VMEM((3, GSZ, H), jnp.float32),
            pltpu.VMEM_SHARED((NPAD, H), jnp.float32),
            [dma, dma, dma],
            [dma, dma, dma],
        ],
    )
    def sc_scatter(m_hbm, idx_hbm, accin_hbm, out_hbm, idx_v, rows_v, acc,
                   isems, rsems):
        c = lax.axis_index("c")
        s = lax.axis_index("s")
        w = c * NS + s

        def load(g, slot):
            pltpu.async_copy(idx_hbm.at[pl.ds(g * GSZ, GSZ)],
                             idx_v.at[slot], isems[slot])
            pltpu.async_copy(m_hbm.at[pl.ds(g * GSZ, GSZ), :],
                             rows_v.at[slot], rsems[slot])

        def load_wait(g, slot):
            pltpu.make_async_copy(idx_hbm.at[pl.ds(g * GSZ, GSZ)],
                                  idx_v.at[slot], isems[slot]).wait()
            pltpu.make_async_copy(m_hbm.at[pl.ds(g * GSZ, GSZ), :],
                                  rows_v.at[slot], rsems[slot]).wait()

        pltpu.sync_copy(accin_hbm.at[c, pl.ds(s * RPT, RPT), :],
                        acc.at[pl.ds(s * RPT, RPT), :])

        @pl.when(w < ngrp)
        def _():
            load(w, 0)

        @pl.when(w + NW < ngrp)
        def _():
            load(w + NW, 1)

        plsc.subcore_barrier()

        def body(i, carry):
            g = w + i * NW
            gn = g + 2 * NW

            for cb in (0, 1, 2):
                @pl.when(lax.rem(i, 3) == cb)
                def _():
                    @pl.when(gn < ngrp)
                    def _():
                        load(gn, (cb + 2) % 3)

                    @pl.when(g < ngrp)
                    def _():
                        load_wait(g, cb)
                        pltpu.sync_copy(rows_v.at[cb], acc.at[idx_v.at[cb]],
                                        add=True)

            return carry

        lax.fori_loop(0, nseq, body, 0)
        plsc.subcore_barrier()
        pltpu.sync_copy(acc.at[pl.ds(s * RPT, RPT), :],
                        out_hbm.at[c, pl.ds(s * RPT, RPT), :])

    return sc_gather, sc_scatter


BN = 2000
BE = 3200


def _xw_body(x_ref, w_ref, b_ref, o_ref):
    o_ref[...] = jnp.dot(x_ref[...], w_ref[...],
                         preferred_element_type=jnp.float32) + b_ref[...]


def _edge_body(xg_ref, ea_ref, w1b_ref, w2_ref, b2_ref, w3_ref, b3_ref, o_ref):
    m1 = jnp.maximum(
        xg_ref[...] + jnp.dot(ea_ref[...], w1b_ref[...],
                              preferred_element_type=jnp.float32), 0.0)
    m2 = jnp.maximum(
        jnp.dot(m1, w2_ref[...], preferred_element_type=jnp.float32)
        + b2_ref[...], 0.0)
    o_ref[...] = (jnp.dot(m2, w3_ref[...], preferred_element_type=jnp.float32)
                  + b3_ref[...])


def _node_body(x_ref, p_ref, w1a_ref, w1b_ref, b1_ref, w2_ref, b2_ref,
               w3_ref, b3_ref, o_ref):
    agg = p_ref[0] + p_ref[1]
    h1 = jnp.maximum(
        jnp.dot(x_ref[...], w1a_ref[...], preferred_element_type=jnp.float32)
        + jnp.dot(agg, w1b_ref[...], preferred_element_type=jnp.float32)
        + b1_ref[...], 0.0)
    h2 = jnp.maximum(
        jnp.dot(h1, w2_ref[...], preferred_element_type=jnp.float32)
        + b2_ref[...], 0.0)
    o_ref[...] = (jnp.dot(h2, w3_ref[...], preferred_element_type=jnp.float32)
                  + b3_ref[...])


def _w_spec(r, c_):
    return pl.BlockSpec((r, c_), lambda i: (0, 0))


def kernel(x, edge_index, edge_attr, u, batch, Wm1, bm1, Wm2, bm2, Wm3, bm3,
           Wn1, bn1, Wn2, bn2, Wn3, bn3):
    send_idx = edge_index[0]
    rec_idx = edge_index[1]
    b2 = bm2.reshape(1, H)
    b3 = bm3.reshape(1, H)
    n1 = bn1.reshape(1, H)
    n2 = bn2.reshape(1, H)
    n3 = bn3.reshape(1, H)

    xw = pl.pallas_call(
        _xw_body,
        grid=(N // BN,),
        in_specs=[pl.BlockSpec((BN, F), lambda i: (i, 0)),
                  _w_spec(F, H), _w_spec(1, H)],
        out_specs=pl.BlockSpec((BN, H), lambda i: (i, 0)),
        out_shape=jax.ShapeDtypeStruct((N, H), jnp.float32),
    )(x, Wm1[:F], bm1.reshape(1, H))

    sc_gather, sc_scatter = _build_sc_kernels(ECH // GSZ)

    edge_mlp = pl.pallas_call(
        _edge_body,
        grid=(ECH // BE,),
        in_specs=[pl.BlockSpec((BE, H), lambda i: (i, 0)),
                  pl.BlockSpec((BE, H), lambda i: (i, 0)),
                  _w_spec(H, H), _w_spec(H, H), _w_spec(1, H),
                  _w_spec(H, H), _w_spec(1, H)],
        out_specs=pl.BlockSpec((BE, H), lambda i: (i, 0)),
        out_shape=jax.ShapeDtypeStruct((ECH, H), jnp.float32),
    )

    partials = jnp.zeros((NC, NPAD, H), jnp.float32)
    for k in range(K):
        sl = slice(k * ECH, (k + 1) * ECH)
        xg_k = sc_gather(xw, send_idx[sl])
        m3_k = edge_mlp(xg_k, edge_attr[sl], Wm1[F:], Wm2, b2, Wm3, b3)
        partials = sc_scatter(m3_k, rec_idx[sl], partials)
    partials = partials[:, :N, :]

    out = pl.pallas_call(
        _node_body,
        grid=(N // BN,),
        in_specs=([pl.BlockSpec((BN, F), lambda i: (i, 0)),
                   pl.BlockSpec((NC, BN, H), lambda i: (0, i, 0)),
                   _w_spec(F, H), _w_spec(H, H), _w_spec(1, H),
                   _w_spec(H, H), _w_spec(1, H), _w_spec(H, H),
                   _w_spec(1, H)]),
        out_specs=pl.BlockSpec((BN, H), lambda i: (i, 0)),
        out_shape=jax.ShapeDtypeStruct((N, H), jnp.float32),
    )(x, partials, Wn1[:F], Wn1[F:], n1, Wn2, n2, Wn3, n3)

    return out


# speedup vs baseline: 1.2266x; 1.2266x over previous
import functools

import jax
import jax.numpy as jnp
from jax import lax
from jax.experimental import pallas as pl
from jax.experimental.pallas import tpu as pltpu
from jax.experimental.pallas import tpu_sc as plsc

N = 10000
E = 320000
F = 128
H = 128

NC = 2
NS = 16
NW = NC * NS
GSZ = 128
RPT = 632
NPAD = NS * RPT

K = 4
ECH = E // K


@functools.cache
def _build_sc_kernels(ngrp, base):
    mesh = plsc.VectorSubcoreMesh(core_axis_name="c", subcore_axis_name="s",
                                  num_cores=NC, num_subcores=NS)
    ec = ngrp * GSZ
    nseq = (ngrp + NW - 1) // NW
    nit = (nseq + 1) // 2

    dma = pltpu.SemaphoreType.DMA

    @functools.partial(
        pl.kernel,
        out_type=jax.ShapeDtypeStruct((ec, H), jnp.float32),
        mesh=mesh,
        scratch_types=[
            pltpu.VMEM((4, GSZ), jnp.int32),
            pltpu.VMEM((4, GSZ, H), jnp.float32),
            [dma, dma, dma, dma],
            [dma, dma, dma, dma],
            [dma, dma],
        ],
    )
    def sc_gather(table_hbm, idx_hbm, out_hbm, idx_v, rows_v,
                  isems, wsems, gsems):
        c = lax.axis_index("c")
        s = lax.axis_index("s")
        w = c * NS + s

        def idx_load(g, slot):
            pltpu.async_copy(idx_hbm.at[pl.ds((base + g) * GSZ, GSZ)],
                             idx_v.at[slot], isems[slot])

        def idx_wait(g, slot):
            pltpu.make_async_copy(idx_hbm.at[pl.ds((base + g) * GSZ, GSZ)],
                                  idx_v.at[slot], isems[slot]).wait()

        def wout(g, slot):
            pltpu.async_copy(rows_v.at[slot],
                             out_hbm.at[pl.ds(g * GSZ, GSZ), :], wsems[slot])

        def wout_wait(g, slot):
            pltpu.make_async_copy(rows_v.at[slot],
                                  out_hbm.at[pl.ds(g * GSZ, GSZ), :],
                                  wsems[slot]).wait()

        @pl.when(w < ngrp)
        def _():
            idx_load(w, 0)

        @pl.when(w + NW < ngrp)
        def _():
            idx_load(w + NW, 1)

        def body(i, carry):
            g0 = w + 2 * i * NW
            g1 = g0 + NW

            for p in (0, 1):
                @pl.when(lax.rem(i, 2) == p)
                def _():
                    s0, s1 = 2 * p, 2 * p + 1
                    q0, q1 = 2 * (1 - p), 2 * (1 - p) + 1
                    n0 = g0 + 2 * NW
                    n1 = g0 + 3 * NW

                    @pl.when(n0 < ngrp)
                    def _():
                        idx_load(n0, q0)

                    @pl.when(n1 < ngrp)
                    def _():
                        idx_load(n1, q1)

                    @pl.when(g0 < ngrp)
                    def _():
                        idx_wait(g0, s0)

                        @pl.when(i >= 2)
                        def _():
                            wout_wait(g0, s0)
                        pltpu.async_copy(table_hbm.at[idx_v.at[s0]],
                                         rows_v.at[s0], gsems[0])

                    @pl.when(g1 < ngrp)
                    def _():
                        idx_wait(g1, s1)

                        @pl.when(i >= 2)
                        def _():
                            wout_wait(g1, s1)
                        pltpu.async_copy(table_hbm.at[idx_v.at[s1]],
                                         rows_v.at[s1], gsems[1])

                    @pl.when(g0 < ngrp)
                    def _():
                        pltpu.make_async_copy(table_hbm.at[idx_v.at[s0]],
                                              rows_v.at[s0], gsems[0]).wait()
                        wout(g0, s0)

                    @pl.when(g1 < ngrp)
                    def _():
                        pltpu.make_async_copy(table_hbm.at[idx_v.at[s1]],
                                              rows_v.at[s1], gsems[1]).wait()
                        wout(g1, s1)

            return carry

        lax.fori_loop(0, nit, body, 0)
        for slot in range(4):
            wout_wait(0, slot)

    @functools.partial(
        pl.kernel,
        out_type=jax.ShapeDtypeStruct((NC, NPAD, H), jnp.float32),
        mesh=mesh,
        scratch_types=[
            pltpu.VMEM((3, GSZ), jnp.int32),
            pltpu.VMEM((3, GSZ, H), jnp.float32),
            pltpu.VMEM_SHARED((NPAD, H), jnp.float32),
            [dma, dma, dma],
            [dma, dma, dma],
        ],
    )
    def sc_scatter(m_hbm, idx_hbm, accin_hbm, out_hbm, idx_v, rows_v, acc,
                   isems, rsems):
        c = lax.axis_index("c")
        s = lax.axis_index("s")
        w = c * NS + s

        def load(g, slot):
            pltpu.async_copy(idx_hbm.at[pl.ds((base + g) * GSZ, GSZ)],
                             idx_v.at[slot], isems[slot])
            pltpu.async_copy(m_hbm.at[pl.ds(g * GSZ, GSZ), :],
                             rows_v.at[slot], rsems[slot])

        def load_wait(g, slot):
            pltpu.make_async_copy(idx_hbm.at[pl.ds((base + g) * GSZ, GSZ)],
                                  idx_v.at[slot], isems[slot]).wait()
            pltpu.make_async_copy(m_hbm.at[pl.ds(g * GSZ, GSZ), :],
                                  rows_v.at[slot], rsems[slot]).wait()

        pltpu.sync_copy(accin_hbm.at[c, pl.ds(s * RPT, RPT), :],
                        acc.at[pl.ds(s * RPT, RPT), :])

        @pl.when(w < ngrp)
        def _():
            load(w, 0)

        @pl.when(w + NW < ngrp)
        def _():
            load(w + NW, 1)

        plsc.subcore_barrier()

        def body(i, carry):
            g = w + i * NW
            gn = g + 2 * NW

            for cb in (0, 1, 2):
                @pl.when(lax.rem(i, 3) == cb)
                def _():
                    @pl.when(gn < ngrp)
                    def _():
                        load(gn, (cb + 2) % 3)

                    @pl.when(g < ngrp)
                    def _():
                        load_wait(g, cb)
                        pltpu.sync_copy(rows_v.at[cb], acc.at[idx_v.at[cb]],
                                        add=True)

            return carry

        lax.fori_loop(0, nseq, body, 0)
        plsc.subcore_barrier()
        pltpu.sync_copy(acc.at[pl.ds(s * RPT, RPT), :],
                        out_hbm.at[c, pl.ds(s * RPT, RPT), :])

    return sc_gather, sc_scatter


BN = 2000
BE = 3200


def _xw_body(x_ref, w_ref, b_ref, o_ref):
    o_ref[...] = jnp.dot(x_ref[...], w_ref[...],
                         preferred_element_type=jnp.float32) + b_ref[...]


def _edge_body(xg_ref, ea_ref, w1b_ref, w2_ref, b2_ref, w3_ref, b3_ref, o_ref):
    m1 = jnp.maximum(
        xg_ref[...] + jnp.dot(ea_ref[...], w1b_ref[...],
                              preferred_element_type=jnp.float32), 0.0)
    m2 = jnp.maximum(
        jnp.dot(m1, w2_ref[...], preferred_element_type=jnp.float32)
        + b2_ref[...], 0.0)
    o_ref[...] = (jnp.dot(m2, w3_ref[...], preferred_element_type=jnp.float32)
                  + b3_ref[...])


def _node_body(x_ref, p_ref, w1a_ref, w1b_ref, b1_ref, w2_ref, b2_ref,
               w3_ref, b3_ref, o_ref):
    agg = p_ref[0] + p_ref[1]
    h1 = jnp.maximum(
        jnp.dot(x_ref[...], w1a_ref[...], preferred_element_type=jnp.float32)
        + jnp.dot(agg, w1b_ref[...], preferred_element_type=jnp.float32)
        + b1_ref[...], 0.0)
    h2 = jnp.maximum(
        jnp.dot(h1, w2_ref[...], preferred_element_type=jnp.float32)
        + b2_ref[...], 0.0)
    o_ref[...] = (jnp.dot(h2, w3_ref[...], preferred_element_type=jnp.float32)
                  + b3_ref[...])


def _w_spec(r, c_):
    return pl.BlockSpec((r, c_), lambda i: (0, 0))


def kernel(x, edge_index, edge_attr, u, batch, Wm1, bm1, Wm2, bm2, Wm3, bm3,
           Wn1, bn1, Wn2, bn2, Wn3, bn3):
    send_idx = edge_index[0]
    rec_idx = edge_index[1]
    b2 = bm2.reshape(1, H)
    b3 = bm3.reshape(1, H)
    n1 = bn1.reshape(1, H)
    n2 = bn2.reshape(1, H)
    n3 = bn3.reshape(1, H)

    xw = pl.pallas_call(
        _xw_body,
        grid=(N // BN,),
        in_specs=[pl.BlockSpec((BN, F), lambda i: (i, 0)),
                  _w_spec(F, H), _w_spec(1, H)],
        out_specs=pl.BlockSpec((BN, H), lambda i: (i, 0)),
        out_shape=jax.ShapeDtypeStruct((N, H), jnp.float32),
    )(x, Wm1[:F], bm1.reshape(1, H))

    partials = jnp.zeros((NC, NPAD, H), jnp.float32)
    nbk = ECH // BE
    for k in range(K):
        sc_gather, sc_scatter = _build_sc_kernels(ECH // GSZ,
                                                  k * (ECH // GSZ))
        xg_k = sc_gather(xw, send_idx)
        m3_k = pl.pallas_call(
            _edge_body,
            grid=(nbk,),
            in_specs=[pl.BlockSpec((BE, H), lambda i: (i, 0)),
                      pl.BlockSpec((BE, H), lambda i, k=k: (k * nbk + i, 0)),
                      _w_spec(H, H), _w_spec(H, H), _w_spec(1, H),
                      _w_spec(H, H), _w_spec(1, H)],
            out_specs=pl.BlockSpec((BE, H), lambda i: (i, 0)),
            out_shape=jax.ShapeDtypeStruct((ECH, H), jnp.float32),
        )(xg_k, edge_attr, Wm1[F:], Wm2, b2, Wm3, b3)
        partials = sc_scatter(m3_k, rec_idx, partials)

    out = pl.pallas_call(
        _node_body,
        grid=(N // BN,),
        in_specs=([pl.BlockSpec((BN, F), lambda i: (i, 0)),
                   pl.BlockSpec((NC, BN, H), lambda i: (0, i, 0)),

                   _w_spec(F, H), _w_spec(H, H), _w_spec(1, H),
                   _w_spec(H, H), _w_spec(1, H), _w_spec(H, H),
                   _w_spec(1, H)]),
        out_specs=pl.BlockSpec((BN, H), lambda i: (i, 0)),
        out_shape=jax.ShapeDtypeStruct((N, H), jnp.float32),
    )(x, partials, Wn1[:F], Wn1[F:], n1, Wn2, n2, Wn3, n3)

    return out


# speedup vs baseline: 1.3385x; 1.0912x over previous
import functools

import jax
import jax.numpy as jnp
from jax import lax
from jax.experimental import pallas as pl
from jax.experimental.pallas import tpu as pltpu
from jax.experimental.pallas import tpu_sc as plsc

N = 10000
E = 320000
F = 128
H = 128

NC = 2
NS = 16
NW = NC * NS
GSZ = 128
RPT = 632
NPAD = NS * RPT

K = 4
ECH = E // K


@functools.cache
def _build_sc_kernels(ngrp, base):
    mesh = plsc.VectorSubcoreMesh(core_axis_name="c", subcore_axis_name="s",
                                  num_cores=NC, num_subcores=NS)
    ec = ngrp * GSZ
    nseq = (ngrp + NW - 1) // NW
    nit = (nseq + 1) // 2

    dma = pltpu.SemaphoreType.DMA

    @functools.partial(
        pl.kernel,
        out_type=jax.ShapeDtypeStruct((ec, H), jnp.float32),
        mesh=mesh,
        scratch_types=[
            pltpu.VMEM((3, GSZ), jnp.int32),
            pltpu.VMEM((3, GSZ, H), jnp.float32),
            pltpu.VMEM_SHARED((NPAD, H), jnp.float32),
            [dma, dma, dma],
            [dma, dma, dma],
            dma,
        ],
    )
    def sc_gather(table_hbm, idx_hbm, out_hbm, idx_v, rows_v, tab,
                  isems, wsems, gsem):
        c = lax.axis_index("c")
        s = lax.axis_index("s")
        w = c * NS + s

        def idx_load(g, slot):
            pltpu.async_copy(idx_hbm.at[pl.ds((base + g) * GSZ, GSZ)],
                             idx_v.at[slot], isems[slot])

        def idx_wait(g, slot):
            pltpu.make_async_copy(idx_hbm.at[pl.ds((base + g) * GSZ, GSZ)],
                                  idx_v.at[slot], isems[slot]).wait()

        def wout(g, slot):
            pltpu.async_copy(rows_v.at[slot],
                             out_hbm.at[pl.ds(g * GSZ, GSZ), :], wsems[slot])

        def wout_wait(g, slot):
            pltpu.make_async_copy(rows_v.at[slot],
                                  out_hbm.at[pl.ds(g * GSZ, GSZ), :],
                                  wsems[slot]).wait()

        pltpu.sync_copy(table_hbm.at[pl.ds(s * RPT, RPT), :],
                        tab.at[pl.ds(s * RPT, RPT), :])

        @pl.when(w < ngrp)
        def _():
            idx_load(w, 0)

        @pl.when(w + NW < ngrp)
        def _():
            idx_load(w + NW, 1)

        plsc.subcore_barrier()

        def body(i, carry):
            g = w + i * NW
            gn = g + 2 * NW

            for cb in (0, 1, 2):
                @pl.when(lax.rem(i, 3) == cb)
                def _():
                    @pl.when(gn < ngrp)
                    def _():
                        idx_load(gn, (cb + 2) % 3)

                    @pl.when(g < ngrp)
                    def _():
                        idx_wait(g, cb)

                        @pl.when(i >= 3)
                        def _():
                            wout_wait(g, cb)
                        pltpu.async_copy(tab.at[idx_v.at[cb]],
                                         rows_v.at[cb], gsem).wait()
                        wout(g, cb)

            return carry

        lax.fori_loop(0, nseq, body, 0)
        for slot in range(3):
            wout_wait(0, slot)

    @functools.partial(
        pl.kernel,
        out_type=jax.ShapeDtypeStruct((NC, NPAD, H), jnp.float32),
        mesh=mesh,
        scratch_types=[
            pltpu.VMEM((3, GSZ), jnp.int32),
            pltpu.VMEM((3, GSZ, H), jnp.float32),
            pltpu.VMEM_SHARED((NPAD, H), jnp.float32),
            [dma, dma, dma],
            [dma, dma, dma],
        ],
    )
    def sc_scatter(m_hbm, idx_hbm, accin_hbm, out_hbm, idx_v, rows_v, acc,
                   isems, rsems):
        c = lax.axis_index("c")
        s = lax.axis_index("s")
        w = c * NS + s

        def load(g, slot):
            pltpu.async_copy(idx_hbm.at[pl.ds((base + g) * GSZ, GSZ)],
                             idx_v.at[slot], isems[slot])
            pltpu.async_copy(m_hbm.at[pl.ds(g * GSZ, GSZ), :],
                             rows_v.at[slot], rsems[slot])

        def load_wait(g, slot):
            pltpu.make_async_copy(idx_hbm.at[pl.ds((base + g) * GSZ, GSZ)],
                                  idx_v.at[slot], isems[slot]).wait()
            pltpu.make_async_copy(m_hbm.at[pl.ds(g * GSZ, GSZ), :],
                                  rows_v.at[slot], rsems[slot]).wait()

        pltpu.sync_copy(accin_hbm.at[c, pl.ds(s * RPT, RPT), :],
                        acc.at[pl.ds(s * RPT, RPT), :])

        @pl.when(w < ngrp)
        def _():
            load(w, 0)

        @pl.when(w + NW < ngrp)
        def _():
            load(w + NW, 1)

        plsc.subcore_barrier()

        def body(i, carry):
            g = w + i * NW
            gn = g + 2 * NW

            for cb in (0, 1, 2):
                @pl.when(lax.rem(i, 3) == cb)
                def _():
                    @pl.when(gn < ngrp)
                    def _():
                        load(gn, (cb + 2) % 3)

                    @pl.when(g < ngrp)
                    def _():
                        load_wait(g, cb)
                        pltpu.sync_copy(rows_v.at[cb], acc.at[idx_v.at[cb]],
                                        add=True)

            return carry

        lax.fori_loop(0, nseq, body, 0)
        plsc.subcore_barrier()
        pltpu.sync_copy(acc.at[pl.ds(s * RPT, RPT), :],
                        out_hbm.at[c, pl.ds(s * RPT, RPT), :])

    return sc_gather, sc_scatter


BN = 2000
BE = 3200


def _xw_body(x_ref, w_ref, b_ref, o_ref):
    o_ref[...] = jnp.dot(x_ref[...], w_ref[...],
                         preferred_element_type=jnp.float32) + b_ref[...]


def _edge_body(xg_ref, ea_ref, w1b_ref, w2_ref, b2_ref, w3_ref, b3_ref, o_ref):
    m1 = jnp.maximum(
        xg_ref[...] + jnp.dot(ea_ref[...], w1b_ref[...],
                              preferred_element_type=jnp.float32), 0.0)
    m2 = jnp.maximum(
        jnp.dot(m1, w2_ref[...], preferred_element_type=jnp.float32)
        + b2_ref[...], 0.0)
    o_ref[...] = (jnp.dot(m2, w3_ref[...], preferred_element_type=jnp.float32)
                  + b3_ref[...])


def _node_body(x_ref, p_ref, w1a_ref, w1b_ref, b1_ref, w2_ref, b2_ref,
               w3_ref, b3_ref, o_ref):
    agg = p_ref[0] + p_ref[1]
    h1 = jnp.maximum(
        jnp.dot(x_ref[...], w1a_ref[...], preferred_element_type=jnp.float32)
        + jnp.dot(agg, w1b_ref[...], preferred_element_type=jnp.float32)
        + b1_ref[...], 0.0)
    h2 = jnp.maximum(
        jnp.dot(h1, w2_ref[...], preferred_element_type=jnp.float32)
        + b2_ref[...], 0.0)
    o_ref[...] = (jnp.dot(h2, w3_ref[...], preferred_element_type=jnp.float32)
                  + b3_ref[...])


def _w_spec(r, c_):
    return pl.BlockSpec((r, c_), lambda i: (0, 0))


def kernel(x, edge_index, edge_attr, u, batch, Wm1, bm1, Wm2, bm2, Wm3, bm3,
           Wn1, bn1, Wn2, bn2, Wn3, bn3):
    send_idx = edge_index[0]
    rec_idx = edge_index[1]
    b2 = bm2.reshape(1, H)
    b3 = bm3.reshape(1, H)
    n1 = bn1.reshape(1, H)
    n2 = bn2.reshape(1, H)
    n3 = bn3.reshape(1, H)

    BW = NPAD // 8
    xw = pl.pallas_call(
        _xw_body,
        grid=(8,),
        in_specs=[pl.BlockSpec((BW, F), lambda i: (i, 0)),
                  _w_spec(F, H), _w_spec(1, H)],
        out_specs=pl.BlockSpec((BW, H), lambda i: (i, 0)),
        out_shape=jax.ShapeDtypeStruct((NPAD, H), jnp.float32),
    )(x, Wm1[:F], bm1.reshape(1, H))

    partials = jnp.zeros((NC, NPAD, H), jnp.float32)
    nbk = ECH // BE
    for k in range(K):
        sc_gather, sc_scatter = _build_sc_kernels(ECH // GSZ,
                                                  k * (ECH // GSZ))
        xg_k = sc_gather(xw, send_idx)
        m3_k = pl.pallas_call(
            _edge_body,
            grid=(nbk,),
            in_specs=[pl.BlockSpec((BE, H), lambda i: (i, 0)),
                      pl.BlockSpec((BE, H), lambda i, k=k: (k * nbk + i, 0)),
                      _w_spec(H, H), _w_spec(H, H), _w_spec(1, H),
                      _w_spec(H, H), _w_spec(1, H)],
            out_specs=pl.BlockSpec((BE, H), lambda i: (i, 0)),
            out_shape=jax.ShapeDtypeStruct((ECH, H), jnp.float32),
        )(xg_k, edge_attr, Wm1[F:], Wm2, b2, Wm3, b3)
        partials = sc_scatter(m3_k, rec_idx, partials)

    out = pl.pallas_call(
        _node_body,
        grid=(N // BN,),
        in_specs=([pl.BlockSpec((BN, F), lambda i: (i, 0)),
                   pl.BlockSpec((NC, BN, H), lambda i: (0, i, 0)),

                   _w_spec(F, H), _w_spec(H, H), _w_spec(1, H),
                   _w_spec(H, H), _w_spec(1, H), _w_spec(H, H),
                   _w_spec(1, H)]),
        out_specs=pl.BlockSpec((BN, H), lambda i: (i, 0)),
        out_shape=jax.ShapeDtypeStruct((N, H), jnp.float32),
    )(x, partials, Wn1[:F], Wn1[F:], n1, Wn2, n2, Wn3, n3)

    return out


# speedup vs baseline: 1.3896x; 1.0382x over previous
import functools

import jax
import jax.numpy as jnp
from jax import lax
from jax.experimental import pallas as pl
from jax.experimental.pallas import tpu as pltpu
from jax.experimental.pallas import tpu_sc as plsc

N = 10000
E = 320000
F = 128
H = 128

NC = 2
NS = 16
NW = NC * NS
GSZ = 128
RPT = 632
NPAD = NS * RPT

K = 4
ECH = E // K


@functools.cache
def _build_sc_kernels(ngrp, base):
    mesh = plsc.VectorSubcoreMesh(core_axis_name="c", subcore_axis_name="s",
                                  num_cores=NC, num_subcores=NS)
    ec = ngrp * GSZ
    nseq = (ngrp + NW - 1) // NW
    nit = (nseq + 1) // 2

    dma = pltpu.SemaphoreType.DMA

    @functools.partial(
        pl.kernel,
        out_type=jax.ShapeDtypeStruct((ec, H), jnp.float32),
        mesh=mesh,
        scratch_types=[
            pltpu.VMEM((3, GSZ), jnp.int32),
            pltpu.VMEM((3, GSZ, H), jnp.float32),
            pltpu.VMEM_SHARED((NPAD, H), jnp.float32),
            [dma, dma, dma],
            [dma, dma, dma],
            dma,
        ],
    )
    def sc_gather(table_hbm, idx_hbm, out_hbm, idx_v, rows_v, tab,
                  isems, wsems, gsem):
        c = lax.axis_index("c")
        s = lax.axis_index("s")
        w = c * NS + s

        def idx_load(g, slot):
            pltpu.async_copy(idx_hbm.at[0, pl.ds((base + g) * GSZ, GSZ)],
                             idx_v.at[slot], isems[slot])

        def idx_wait(g, slot):
            pltpu.make_async_copy(idx_hbm.at[0, pl.ds((base + g) * GSZ, GSZ)],
                                  idx_v.at[slot], isems[slot]).wait()

        def wout(g, slot):
            pltpu.async_copy(rows_v.at[slot],
                             out_hbm.at[pl.ds(g * GSZ, GSZ), :], wsems[slot])

        def wout_wait(g, slot):
            pltpu.make_async_copy(rows_v.at[slot],
                                  out_hbm.at[pl.ds(g * GSZ, GSZ), :],
                                  wsems[slot]).wait()

        pltpu.sync_copy(table_hbm.at[pl.ds(s * RPT, RPT), :],
                        tab.at[pl.ds(s * RPT, RPT), :])

        @pl.when(w < ngrp)
        def _():
            idx_load(w, 0)

        @pl.when(w + NW < ngrp)
        def _():
            idx_load(w + NW, 1)

        plsc.subcore_barrier()

        def body(i, carry):
            g = w + i * NW
            gn = g + 2 * NW

            for cb in (0, 1, 2):
                @pl.when(lax.rem(i, 3) == cb)
                def _():
                    @pl.when(gn < ngrp)
                    def _():
                        idx_load(gn, (cb + 2) % 3)

                    @pl.when(g < ngrp)
                    def _():
                        idx_wait(g, cb)

                        @pl.when(i >= 3)
                        def _():
                            wout_wait(g, cb)
                        pltpu.async_copy(tab.at[idx_v.at[cb]],
                                         rows_v.at[cb], gsem).wait()
                        wout(g, cb)

            return carry

        lax.fori_loop(0, nseq, body, 0)
        for slot in range(3):
            wout_wait(0, slot)

    @functools.partial(
        pl.kernel,
        out_type=jax.ShapeDtypeStruct((NC, NPAD, H), jnp.float32),
        mesh=mesh,
        scratch_types=[
            pltpu.VMEM((3, GSZ), jnp.int32),
            pltpu.VMEM((3, GSZ, H), jnp.float32),
            pltpu.VMEM_SHARED((NPAD, H), jnp.float32),
            [dma, dma, dma],
            [dma, dma, dma],
        ],
    )
    def sc_scatter(m_hbm, idx_hbm, accin_hbm, out_hbm, idx_v, rows_v, acc,
                   isems, rsems):
        c = lax.axis_index("c")
        s = lax.axis_index("s")
        w = c * NS + s

        def load(g, slot):
            pltpu.async_copy(idx_hbm.at[1, pl.ds((base + g) * GSZ, GSZ)],
                             idx_v.at[slot], isems[slot])
            pltpu.async_copy(m_hbm.at[pl.ds(g * GSZ, GSZ), :],
                             rows_v.at[slot], rsems[slot])

        def load_wait(g, slot):
            pltpu.make_async_copy(idx_hbm.at[1, pl.ds((base + g) * GSZ, GSZ)],
                                  idx_v.at[slot], isems[slot]).wait()
            pltpu.make_async_copy(m_hbm.at[pl.ds(g * GSZ, GSZ), :],
                                  rows_v.at[slot], rsems[slot]).wait()

        pltpu.sync_copy(accin_hbm.at[c, pl.ds(s * RPT, RPT), :],
                        acc.at[pl.ds(s * RPT, RPT), :])

        @pl.when(w < ngrp)
        def _():
            load(w, 0)

        @pl.when(w + NW < ngrp)
        def _():
            load(w + NW, 1)

        plsc.subcore_barrier()

        def body(i, carry):
            g = w + i * NW
            gn = g + 2 * NW

            for cb in (0, 1, 2):
                @pl.when(lax.rem(i, 3) == cb)
                def _():
                    @pl.when(gn < ngrp)
                    def _():
                        load(gn, (cb + 2) % 3)

                    @pl.when(g < ngrp)
                    def _():
                        load_wait(g, cb)
                        pltpu.sync_copy(rows_v.at[cb], acc.at[idx_v.at[cb]],
                                        add=True)

            return carry

        lax.fori_loop(0, nseq, body, 0)
        plsc.subcore_barrier()
        pltpu.sync_copy(acc.at[pl.ds(s * RPT, RPT), :],
                        out_hbm.at[c, pl.ds(s * RPT, RPT), :])

    return sc_gather, sc_scatter


BN = 2000
BE = 3200


def _xw_body(x_ref, w_ref, b_ref, o_ref):
    o_ref[...] = jnp.dot(x_ref[...], w_ref[...],
                         preferred_element_type=jnp.float32) + b_ref[...]


def _bdot(a, b):
    return jnp.dot(a.astype(jnp.bfloat16), b.astype(jnp.bfloat16),
                   preferred_element_type=jnp.float32)


def _edge_body(xg_ref, ea_ref, w1b_ref, w2_ref, b2_ref, w3_ref, b3_ref, o_ref):
    m1 = jnp.maximum(xg_ref[...] + _bdot(ea_ref[...], w1b_ref[...]), 0.0)
    m2 = jnp.maximum(_bdot(m1, w2_ref[...]) + b2_ref[...], 0.0)
    o_ref[...] = _bdot(m2, w3_ref[...]) + b3_ref[...]


def _node_body(x_ref, p_ref, w1a_ref, w1b_ref, b1_ref, w2_ref, b2_ref,
               w3_ref, b3_ref, o_ref):
    agg = p_ref[0] + p_ref[1]
    h1 = jnp.maximum(
        jnp.dot(x_ref[...], w1a_ref[...], preferred_element_type=jnp.float32)
        + jnp.dot(agg, w1b_ref[...], preferred_element_type=jnp.float32)
        + b1_ref[...], 0.0)
    h2 = jnp.maximum(
        jnp.dot(h1, w2_ref[...], preferred_element_type=jnp.float32)
        + b2_ref[...], 0.0)
    o_ref[...] = (jnp.dot(h2, w3_ref[...], preferred_element_type=jnp.float32)
                  + b3_ref[...])


def _w_spec(r, c_):
    return pl.BlockSpec((r, c_), lambda i: (0, 0))


def kernel(x, edge_index, edge_attr, u, batch, Wm1, bm1, Wm2, bm2, Wm3, bm3,
           Wn1, bn1, Wn2, bn2, Wn3, bn3):
    b2 = bm2.reshape(1, H)
    b3 = bm3.reshape(1, H)
    n1 = bn1.reshape(1, H)
    n2 = bn2.reshape(1, H)
    n3 = bn3.reshape(1, H)

    BW = NPAD // 8
    xw = pl.pallas_call(
        _xw_body,
        grid=(8,),
        in_specs=[pl.BlockSpec((BW, F), lambda i: (i, 0)),
                  _w_spec(F, H), _w_spec(1, H)],
        out_specs=pl.BlockSpec((BW, H), lambda i: (i, 0)),
        out_shape=jax.ShapeDtypeStruct((NPAD, H), jnp.float32),
    )(x, Wm1[:F], bm1.reshape(1, H))

    partials = jnp.zeros((NC, NPAD, H), jnp.float32)
    nbk = ECH // BE
    for k in range(K):
        sc_gather, sc_scatter = _build_sc_kernels(ECH // GSZ,
                                                  k * (ECH // GSZ))
        xg_k = sc_gather(xw, edge_index)
        m3_k = pl.pallas_call(
            _edge_body,
            grid=(nbk,),
            in_specs=[pl.BlockSpec((BE, H), lambda i: (i, 0)),
                      pl.BlockSpec((BE, H), lambda i, k=k: (k * nbk + i, 0)),
                      _w_spec(H, H), _w_spec(H, H), _w_spec(1, H),
                      _w_spec(H, H), _w_spec(1, H)],
            out_specs=pl.BlockSpec((BE, H), lambda i: (i, 0)),
            out_shape=jax.ShapeDtypeStruct((ECH, H), jnp.float32),
        )(xg_k, edge_attr, Wm1[F:], Wm2, b2, Wm3, b3)
        partials = sc_scatter(m3_k, edge_index, partials)

    out = pl.pallas_call(
        _node_body,
        grid=(N // BN,),
        in_specs=([pl.BlockSpec((BN, F), lambda i: (i, 0)),
                   pl.BlockSpec((NC, BN, H), lambda i: (0, i, 0)),

                   _w_spec(F, H), _w_spec(H, H), _w_spec(1, H),
                   _w_spec(H, H), _w_spec(1, H), _w_spec(H, H),
                   _w_spec(1, H)]),
        out_specs=pl.BlockSpec((BN, H), lambda i: (i, 0)),
        out_shape=jax.ShapeDtypeStruct((N, H), jnp.float32),
    )(x, partials, Wn1[:F], Wn1[F:], n1, Wn2, n2, Wn3, n3)

    return out
